# baseline (speedup 1.0000x reference)
import jax
import jax.numpy as jnp
from jax.experimental import pallas as pl
from jax.experimental.pallas import tpu as pltpu

CAPACITY = 262144
OBS_DIM = 128
BATCH = 16384
TAIL = CAPACITY - BATCH
SB = BATCH // 128
SC = CAPACITY // 128
STAIL = SC - SB


def _body(st, ac, ns, rw, stm, acm, nsm, rwm, ost, oac, ons, orw, *sems):
    pairs = (
        (st, ost.at[pl.ds(0, BATCH)]),
        (stm.at[pl.ds(BATCH, TAIL)], ost.at[pl.ds(BATCH, TAIL)]),
        (ac, oac.at[pl.ds(0, SB)]),
        (acm.at[pl.ds(SB, STAIL)], oac.at[pl.ds(SB, STAIL)]),
        (ns, ons.at[pl.ds(0, BATCH)]),
        (nsm.at[pl.ds(BATCH, TAIL)], ons.at[pl.ds(BATCH, TAIL)]),
        (rw, orw.at[pl.ds(0, SB)]),
        (rwm.at[pl.ds(SB, STAIL)], orw.at[pl.ds(SB, STAIL)]),
    )
    copies = [pltpu.make_async_copy(s, d, sem) for (s, d), sem in zip(pairs, sems)]
    for c in copies:
        c.start()
    for c in copies:
        c.wait()


def kernel(states, actions, next_states, rewards, states_mem, next_states_mem, actions_mem, rewards_mem):
    ac2 = actions.reshape(SB, 128)
    rw2 = rewards.reshape(SB, 128)
    acm2 = actions_mem.reshape(SC, 128)
    rwm2 = rewards_mem.reshape(SC, 128)

    any_spec = pl.BlockSpec(memory_space=pl.ANY)

    out_st, out_ac2, out_ns, out_rw2 = pl.pallas_call(
        _body,
        in_specs=[any_spec] * 8,
        out_specs=[any_spec] * 4,
        out_shape=[
            jax.ShapeDtypeStruct((CAPACITY, OBS_DIM), jnp.float32),
            jax.ShapeDtypeStruct((SC, 128), jnp.int32),
            jax.ShapeDtypeStruct((CAPACITY, OBS_DIM), jnp.float32),
            jax.ShapeDtypeStruct((SC, 128), jnp.float32),
        ],
        scratch_shapes=[pltpu.SemaphoreType.DMA] * 8,
    )(states, ac2, next_states, rw2, states_mem, acm2, next_states_mem, rwm2)

    return (out_st, out_ac2.reshape(CAPACITY), out_ns, out_rw2.reshape(CAPACITY))


# speedup vs baseline: 48.5768x; 48.5768x over previous
import jax
import jax.numpy as jnp
from jax.experimental import pallas as pl

CAPACITY = 262144
OBS_DIM = 128
BATCH = 16384

ROWS = 8192
GRID = CAPACITY // ROWS
NB_BATCH = BATCH // ROWS
SROWS = ROWS // 128


def _body(st, ac, ns, rw, stm, acm, nsm, rwm, ost, oac, ons, orw):
    i = pl.program_id(0)

    @pl.when(i < NB_BATCH)
    def _():
        ost[...] = st[...]
        oac[...] = ac[...]
        ons[...] = ns[...]
        orw[...] = rw[...]

    @pl.when(i >= NB_BATCH)
    def _():
        ost[...] = stm[...]
        oac[...] = acm[...]
        ons[...] = nsm[...]
        orw[...] = rwm[...]


def kernel(states, actions, next_states, rewards, states_mem, next_states_mem, actions_mem, rewards_mem):
    ac2 = actions.reshape(BATCH // 128, 128)
    rw2 = rewards.reshape(BATCH // 128, 128)
    acm2 = actions_mem.reshape(CAPACITY // 128, 128)
    rwm2 = rewards_mem.reshape(CAPACITY // 128, 128)

    big = pl.BlockSpec((ROWS, OBS_DIM), lambda i: (i, 0))
    small = pl.BlockSpec((SROWS, 128), lambda i: (i, 0))
    big_mem = pl.BlockSpec((ROWS, OBS_DIM), lambda i: (jnp.maximum(i, NB_BATCH), 0))
    small_mem = pl.BlockSpec((SROWS, 128), lambda i: (jnp.maximum(i, NB_BATCH), 0))
    big_batch = pl.BlockSpec((ROWS, OBS_DIM), lambda i: (jnp.minimum(i, NB_BATCH - 1), 0))
    small_batch = pl.BlockSpec((SROWS, 128), lambda i: (jnp.minimum(i, NB_BATCH - 1), 0))

    out_st, out_ac2, out_ns, out_rw2 = pl.pallas_call(
        _body,
        grid=(GRID,),
        in_specs=[big_batch, small_batch, big_batch, small_batch,
                  big_mem, small_mem, big_mem, small_mem],
        out_specs=[big, small, big, small],
        out_shape=[
            jax.ShapeDtypeStruct((CAPACITY, OBS_DIM), jnp.float32),
            jax.ShapeDtypeStruct((CAPACITY // 128, 128), jnp.int32),
            jax.ShapeDtypeStruct((CAPACITY, OBS_DIM), jnp.float32),
            jax.ShapeDtypeStruct((CAPACITY // 128, 128), jnp.float32),
        ],
    )(states, ac2, next_states, rw2, states_mem, acm2, next_states_mem, rwm2)

    return (out_st, out_ac2.reshape(CAPACITY), out_ns, out_rw2.reshape(CAPACITY))


# speedup vs baseline: 48.6280x; 1.0011x over previous
import jax
import jax.numpy as jnp
from jax.experimental import pallas as pl
from jax.experimental.pallas import tpu as pltpu

CAPACITY = 262144
OBS_DIM = 128
BATCH = 16384

ROWS = 8192
GRID = CAPACITY // ROWS
NB_BATCH = BATCH // ROWS
SROWS = ROWS // 128


def _body(st, ac, ns, rw, stm, acm, nsm, rwm, ost, oac, ons, orw):
    i = pl.program_id(0)

    @pl.when(i < NB_BATCH)
    def _():
        ost[...] = st[...]
        oac[...] = ac[...]
        ons[...] = ns[...]
        orw[...] = rw[...]

    @pl.when(i >= NB_BATCH)
    def _():
        ost[...] = stm[...]
        oac[...] = acm[...]
        ons[...] = nsm[...]
        orw[...] = rwm[...]


def kernel(states, actions, next_states, rewards, states_mem, next_states_mem, actions_mem, rewards_mem):
    ac2 = actions.reshape(BATCH // 128, 128)
    rw2 = rewards.reshape(BATCH // 128, 128)
    acm2 = actions_mem.reshape(CAPACITY // 128, 128)
    rwm2 = rewards_mem.reshape(CAPACITY // 128, 128)

    big = pl.BlockSpec((ROWS, OBS_DIM), lambda i: (i, 0))
    small = pl.BlockSpec((SROWS, 128), lambda i: (i, 0))
    big_mem = pl.BlockSpec((ROWS, OBS_DIM), lambda i: (jnp.maximum(i, NB_BATCH), 0))
    small_mem = pl.BlockSpec((SROWS, 128), lambda i: (jnp.maximum(i, NB_BATCH), 0))
    big_batch = pl.BlockSpec((ROWS, OBS_DIM), lambda i: (jnp.minimum(i, NB_BATCH - 1), 0))
    small_batch = pl.BlockSpec((SROWS, 128), lambda i: (jnp.minimum(i, NB_BATCH - 1), 0))

    out_st, out_ac2, out_ns, out_rw2 = pl.pallas_call(
        _body,
        grid=(GRID,),
        in_specs=[big_batch, small_batch, big_batch, small_batch,
                  big_mem, small_mem, big_mem, small_mem],
        out_specs=[big, small, big, small],
        out_shape=[
            jax.ShapeDtypeStruct((CAPACITY, OBS_DIM), jnp.float32),
            jax.ShapeDtypeStruct((CAPACITY // 128, 128), jnp.int32),
            jax.ShapeDtypeStruct((CAPACITY, OBS_DIM), jnp.float32),
            jax.ShapeDtypeStruct((CAPACITY // 128, 128), jnp.float32),
        ],
        compiler_params=pltpu.CompilerParams(dimension_semantics=("parallel",)),
    )(states, ac2, next_states, rw2, states_mem, acm2, next_states_mem, rwm2)

    return (out_st, out_ac2.reshape(CAPACITY), out_ns, out_rw2.reshape(CAPACITY))
